# Initial kernel scaffold; baseline (speedup 1.0000x reference)
import jax
import jax.numpy as jnp
from jax.experimental import pallas as pl


def kernel(D_ca, Angle_cab, id3_reduce_ca, Kidx):
    raise NotImplementedError("write your pallas kernel here")


# speedup vs baseline: 7.1759x; 7.1759x over previous
import functools

import numpy as np
import jax
import jax.numpy as jnp
from jax import lax
from jax.experimental import pallas as pl
from jax.experimental.pallas import tpu as pltpu
from jax.experimental.pallas import tpu_sc as plsc

NUM_SPHERICAL = 7
NUM_RADIAL = 6
CUTOFF = 5.0
ENVELOPE_EXPONENT = 5
INV_CUTOFF = 1.0 / CUTOFF
NORM_CONST = INV_CUTOFF ** 1.5
N_EDGES = 160000
N_TRIPLETS = 640000
NLN = NUM_SPHERICAL * NUM_RADIAL
DPAD = 48


def _jn_np(r, n):
    r = np.asarray(r, dtype=np.float64)
    j0 = np.sin(r) / r
    if n == 0:
        return j0
    j1 = np.sin(r) / r ** 2 - np.cos(r) / r
    for l in range(1, n):
        j0, j1 = j1, (2 * l + 1) / r * j1 - j0
    return j1


def _bisect_zero(n, a, b, iters=100):
    fa = _jn_np(a, n)
    for _ in range(iters):
        m = 0.5 * (a + b)
        fm = _jn_np(m, n)
        if fa * fm <= 0.0:
            b = m
        else:
            a = m
            fa = fm
    return 0.5 * (a + b)


def _jn_zeros(n, k):
    zerosj = np.zeros((n, k))
    zerosj[0] = np.arange(1, k + 1) * np.pi
    points = np.arange(1, k + n) * np.pi
    racines = np.zeros(k + n - 1)
    for i in range(1, n):
        for j in range(k + n - 1 - i):
            racines[j] = _bisect_zero(i, points[j], points[j + 1])
        points = racines.copy()
        zerosj[i][:k] = racines[:k]
    return zerosj


_ZEROS = _jn_zeros(NUM_SPHERICAL, NUM_RADIAL)
_NORM = np.zeros((NUM_SPHERICAL, NUM_RADIAL))
for _l in range(NUM_SPHERICAL):
    for _n in range(NUM_RADIAL):
        _NORM[_l, _n] = 1.0 / np.sqrt(0.5 * _jn_np(_ZEROS[_l, _n], _l + 1) ** 2)

_Z_COL = np.ones((DPAD, 1), dtype=np.float32)
_Z_COL[:NLN, 0] = _ZEROS.reshape(-1)
_NORM_COL = np.zeros((DPAD, 1), dtype=np.float32)
_NORM_COL[:NLN, 0] = (_NORM * NORM_CONST).reshape(-1)
_L_COL = np.zeros((DPAD, 1), dtype=np.int32)
_L_COL[:NLN, 0] = np.repeat(np.arange(NUM_SPHERICAL), NUM_RADIAL)

_SPH_COEF = np.sqrt((2 * np.arange(NUM_SPHERICAL) + 1) / (4.0 * np.pi))

_P = ENVELOPE_EXPONENT + 1
_ENV_A = -(_P + 1) * (_P + 2) / 2.0
_ENV_B = float(_P * (_P + 2))
_ENV_C = -_P * (_P + 1) / 2.0

_NC = 2
_NS = 16
_NW = _NC * _NS
_ROWS_PER_W = N_TRIPLETS // _NW
_CH = 2000
_NCHUNK = _ROWS_PER_W // _CH


_W_RBF = 3200


def _rbf_body(z_ref, n_ref, d_ref, o_ref):
    zcol = z_ref[...]
    ncol = n_ref[...]
    lcol = lax.broadcasted_iota(jnp.int32, (DPAD, 1), 0) // NUM_RADIAL
    d = d_ref[...] * INV_CUTOFF
    inv_d = 1.0 / d
    d2 = d * d
    d4 = d2 * d2
    d5 = d4 * d
    env = inv_d + _ENV_A * d5 + _ENV_B * d5 * d + _ENV_C * d5 * d2
    env = jnp.where(d < 1.0, env, 0.0)
    x = zcol * d
    inv_x = 1.0 / x
    s = jnp.sin(x)
    c = jnp.cos(x)
    j0 = s * inv_x
    j1 = (j0 - c) * inv_x
    res = jnp.where(lcol == 0, j0, 0.0)
    res = jnp.where(lcol == 1, j1, res)
    for i in range(1, NUM_SPHERICAL - 1):
        j0, j1 = j1, (2 * i + 1) * inv_x * j1 - j0
        res = jnp.where(lcol == i + 1, j1, res)
    o_ref[...] = res * ncol * env


_W_SPH = 6400


def _sph_body(a_ref, o_ref):
    ct = jnp.cos(a_ref[...])
    rows = [float(_SPH_COEF[0]) * jnp.ones_like(ct), float(_SPH_COEF[1]) * ct]
    p0 = jnp.ones_like(ct)
    p1 = ct
    for i in range(1, NUM_SPHERICAL - 1):
        p0, p1 = p1, ((2 * i + 1) * ct * p1 - i * p0) / (i + 1.0)
        rows.append(float(_SPH_COEF[i + 1]) * p1)
    rows.append(jnp.zeros_like(ct))
    o_ref[...] = jnp.concatenate(rows, axis=0)


def _sc_gather(table, idx):
    mesh = plsc.VectorSubcoreMesh(core_axis_name="c", subcore_axis_name="s")

    @functools.partial(
        pl.kernel,
        mesh=mesh,
        compiler_params=pltpu.CompilerParams(use_tc_tiling_on_sc=False),
        out_type=jax.ShapeDtypeStruct((N_TRIPLETS, DPAD), jnp.float32),
        scratch_types=[
            pltpu.VMEM((_CH,), jnp.int32),
            pltpu.VMEM((_CH, DPAD), jnp.float32),
            pltpu.SemaphoreType.DMA,
        ],
    )
    def k(table_hbm, idx_hbm, out_hbm, idx_v, rows_v, sem):
        wid = lax.axis_index("s") * _NC + lax.axis_index("c")
        base = wid * _ROWS_PER_W

        def body(i, carry):
            off = base + i * _CH
            pltpu.sync_copy(idx_hbm.at[pl.ds(off, _CH)], idx_v)
            pltpu.async_copy(table_hbm.at[idx_v], rows_v, sem).wait()
            pltpu.sync_copy(rows_v, out_hbm.at[pl.ds(off, _CH)])
            return carry

        lax.fori_loop(0, _NCHUNK, body, 0)

    return k(table, idx)


_T_MUL = 5120


def _mul_body(g_ref, s_ref, o_ref):
    g = g_ref[...]
    s = s_ref[...]
    parts = [
        jnp.broadcast_to(s[:, l:l + 1], (_T_MUL, NUM_RADIAL))
        for l in range(NUM_SPHERICAL)
    ]
    sphe = jnp.concatenate(parts, axis=1)
    o_ref[...] = g[:, :NLN] * sphe


def kernel(D_ca, Angle_cab, id3_reduce_ca, Kidx):
    del Kidx
    rbf_t = pl.pallas_call(
        _rbf_body,
        grid=(N_EDGES // _W_RBF,),
        in_specs=[
            pl.BlockSpec((DPAD, 1), lambda i: (0, 0)),
            pl.BlockSpec((DPAD, 1), lambda i: (0, 0)),
            pl.BlockSpec((1, _W_RBF), lambda i: (0, i)),
        ],
        out_specs=pl.BlockSpec((DPAD, _W_RBF), lambda i: (0, i)),
        out_shape=jax.ShapeDtypeStruct((DPAD, N_EDGES), jnp.float32),
    )(jnp.asarray(_Z_COL), jnp.asarray(_NORM_COL), D_ca.reshape(1, N_EDGES))
    table = rbf_t.T

    sph_t = pl.pallas_call(
        _sph_body,
        grid=(N_TRIPLETS // _W_SPH,),
        in_specs=[pl.BlockSpec((1, _W_SPH), lambda i: (0, i))],
        out_specs=pl.BlockSpec((8, _W_SPH), lambda i: (0, i)),
        out_shape=jax.ShapeDtypeStruct((8, N_TRIPLETS), jnp.float32),
    )(Angle_cab.reshape(1, N_TRIPLETS))
    sph = sph_t.T

    gath = _sc_gather(table, id3_reduce_ca)

    out = pl.pallas_call(
        _mul_body,
        grid=(N_TRIPLETS // _T_MUL,),
        in_specs=[
            pl.BlockSpec((_T_MUL, DPAD), lambda i: (i, 0)),
            pl.BlockSpec((_T_MUL, 8), lambda i: (i, 0)),
        ],
        out_specs=pl.BlockSpec((_T_MUL, NLN), lambda i: (i, 0)),
        out_shape=jax.ShapeDtypeStruct((N_TRIPLETS, NLN), jnp.float32),
    )(gath, sph)
    return out
